# baseline (speedup 1.0000x reference)
import functools

import jax
import jax.numpy as jnp
from jax.experimental import pallas as pl
from jax.experimental.pallas import tpu as pltpu

_K = 64


def _matmul_into(x_ref, z_ref, w_ref, s_ref, buf):
    upd = jax.lax.dot_general(
        x_ref[...], w_ref[...], (((1,), (1,)), ((), ())),
        preferred_element_type=jnp.float32)
    upd = upd + jax.lax.dot_general(
        z_ref[...], s_ref[...], (((1,), (1,)), ((), ())),
        preferred_element_type=jnp.float32)
    buf[...] = upd


def _probe(bits, state, mid):
    lo, hi, cnt_lo, cnt_hi = state
    mid = jnp.clip(mid, lo + 1, jnp.maximum(hi - 1, lo + 1))
    cnt = jnp.sum((bits >= mid).astype(jnp.int32), axis=1, keepdims=True)
    ge = cnt >= _K
    return (jnp.where(ge, mid, lo), jnp.where(ge, hi, mid),
            jnp.where(ge, cnt, cnt_lo), jnp.where(ge, cnt_hi, cnt))


def _interp_probe(bits, state):
    lo, hi, cnt_lo, cnt_hi = state
    span = (hi - lo).astype(jnp.float32)
    frac = (cnt_lo - _K).astype(jnp.float32) / jnp.maximum(
        cnt_lo - cnt_hi, 1).astype(jnp.float32)
    return _probe(bits, state, lo + (span * frac).astype(jnp.int32))


def _bisect_probe(bits, state):
    lo, hi, _, _ = state
    return _probe(bits, state, lo + ((hi - lo) >> 1))


def _unconverged(refs):
    lo, hi, cnt_lo, _ = refs
    return jnp.any((cnt_lo[...] != _K) & (hi[...] > lo[...] + 1))


def _store(refs, state):
    for r, v in zip(refs, state):
        r[...] = v


def _load(refs):
    return tuple(r[...] for r in refs)


def _select_store(buf, o_ref, refs):
    upd = buf[...]
    bits = jax.lax.bitcast_convert_type(upd, jnp.int32) & jnp.int32(0x7FFFFFFF)
    rows, cols = upd.shape
    state = (jnp.zeros((rows, 1), jnp.int32),
             jnp.max(bits, axis=1, keepdims=True) + 1,
             jnp.full((rows, 1), cols, jnp.int32),
             jnp.zeros((rows, 1), jnp.int32))
    for _ in range(14):
        state = _interp_probe(bits, state)
    _store(refs, state)

    for extra in (5, 5):
        @pl.when(_unconverged(refs))
        def _(extra=extra):
            st = _load(refs)
            for _ in range(extra):
                st = _interp_probe(bits, st)
            _store(refs, st)

    @pl.when(_unconverged(refs))
    def _():
        st = jax.lax.fori_loop(
            0, 34, lambda _, s: _bisect_probe(bits, s), _load(refs))
        _store(refs, st)

    o_ref[...] = jnp.where(bits >= refs[0][...], upd, 0.0)


def _pipelined_block(x_ref, z_ref, w_ref, s_ref, o_ref, buf, lo_r, hi_r,
                     cl_r, ch_r, *, nblocks):
    i = pl.program_id(0)
    _select_store(buf.at[(i + 1) % 2], o_ref, (lo_r, hi_r, cl_r, ch_r))
    _matmul_into(x_ref, z_ref, w_ref, s_ref, buf.at[i % 2])


@functools.partial(jax.jit, static_argnames=("block_rows",))
def kernel(x, z_prev, W, S, block_rows: int = 256):
    batch, input_dim = x.shape
    code_dim = W.shape[0]
    nblocks = batch // block_rows
    grid = (nblocks + 1,)
    return pl.pallas_call(
        functools.partial(_pipelined_block, nblocks=nblocks),
        grid=grid,
        in_specs=[
            pl.BlockSpec((block_rows, input_dim),
                         lambda i: (jnp.minimum(i, nblocks - 1), 0)),
            pl.BlockSpec((block_rows, code_dim),
                         lambda i: (jnp.minimum(i, nblocks - 1), 0)),
            pl.BlockSpec((code_dim, input_dim), lambda i: (0, 0)),
            pl.BlockSpec((code_dim, code_dim), lambda i: (0, 0)),
        ],
        out_specs=pl.BlockSpec((block_rows, code_dim),
                               lambda i: (jnp.maximum(i - 1, 0), 0)),
        out_shape=jax.ShapeDtypeStruct((batch, code_dim), jnp.float32),
        scratch_shapes=[
            pltpu.VMEM((2, block_rows, code_dim), jnp.float32),
            pltpu.VMEM((block_rows, 1), jnp.int32),
            pltpu.VMEM((block_rows, 1), jnp.int32),
            pltpu.VMEM((block_rows, 1), jnp.int32),
            pltpu.VMEM((block_rows, 1), jnp.int32),
        ],
    )(x, z_prev, W, S)


# speedup vs baseline: 1.3583x; 1.3583x over previous
import functools

import jax
import jax.numpy as jnp
from jax.experimental import pallas as pl
from jax.experimental.pallas import tpu as pltpu

_K = 64


def _matmul_into(x_ref, z_ref, w_ref, s_ref, buf):
    updt = jax.lax.dot_general(
        w_ref[...], x_ref[...], (((1,), (1,)), ((), ())),
        preferred_element_type=jnp.float32)
    updt = updt + jax.lax.dot_general(
        s_ref[...], z_ref[...], (((1,), (1,)), ((), ())),
        preferred_element_type=jnp.float32)
    buf[...] = updt


def _probe(bits, state, mid):
    lo, hi, cnt_lo, cnt_hi = state
    mid = jnp.clip(mid, lo + 1, jnp.maximum(hi - 1, lo + 1))
    cnt = jnp.sum((bits >= mid).astype(jnp.int32), axis=0, keepdims=True)
    ge = cnt >= _K
    return (jnp.where(ge, mid, lo), jnp.where(ge, hi, mid),
            jnp.where(ge, cnt, cnt_lo), jnp.where(ge, cnt_hi, cnt))


def _interp_probe(bits, state):
    lo, hi, cnt_lo, cnt_hi = state
    span = (hi - lo).astype(jnp.float32)
    frac = (cnt_lo - _K).astype(jnp.float32) / jnp.maximum(
        cnt_lo - cnt_hi, 1).astype(jnp.float32)
    return _probe(bits, state, lo + (span * frac).astype(jnp.int32))


def _bisect_probe(bits, state):
    lo, hi, _, _ = state
    return _probe(bits, state, lo + ((hi - lo) >> 1))


def _unconverged(refs):
    lo, hi, cnt_lo, _ = refs
    return jnp.any((cnt_lo[...] != _K) & (hi[...] > lo[...] + 1))


def _store(refs, state):
    for r, v in zip(refs, state):
        r[...] = v


def _load(refs):
    return tuple(r[...] for r in refs)


def _select_store(buf, o_ref, refs):
    updt = buf[...]
    bits = jax.lax.bitcast_convert_type(updt, jnp.int32) & jnp.int32(0x7FFFFFFF)
    cols, rows = updt.shape
    state = (jnp.zeros((1, rows), jnp.int32),
             jnp.max(bits, axis=0, keepdims=True) + 1,
             jnp.full((1, rows), cols, jnp.int32),
             jnp.zeros((1, rows), jnp.int32))
    for _ in range(14):
        state = _interp_probe(bits, state)
    _store(refs, state)

    for extra in (5, 5):
        @pl.when(_unconverged(refs))
        def _(extra=extra):
            st = _load(refs)
            for _ in range(extra):
                st = _interp_probe(bits, st)
            _store(refs, st)

    @pl.when(_unconverged(refs))
    def _():
        st = jax.lax.fori_loop(
            0, 34, lambda _, s: _bisect_probe(bits, s), _load(refs))
        _store(refs, st)

    o_ref[...] = jnp.where(bits >= refs[0][...], updt, 0.0).T


def _pipelined_block(x_ref, z_ref, w_ref, s_ref, o_ref, buf, lo_r, hi_r,
                     cl_r, ch_r, *, nblocks):
    i = pl.program_id(0)
    _select_store(buf.at[(i + 1) % 2], o_ref, (lo_r, hi_r, cl_r, ch_r))
    _matmul_into(x_ref, z_ref, w_ref, s_ref, buf.at[i % 2])


@functools.partial(jax.jit, static_argnames=("block_rows",))
def kernel(x, z_prev, W, S, block_rows: int = 256):
    batch, input_dim = x.shape
    code_dim = W.shape[0]
    nblocks = batch // block_rows
    grid = (nblocks + 1,)
    return pl.pallas_call(
        functools.partial(_pipelined_block, nblocks=nblocks),
        grid=grid,
        in_specs=[
            pl.BlockSpec((block_rows, input_dim),
                         lambda i: (jnp.minimum(i, nblocks - 1), 0)),
            pl.BlockSpec((block_rows, code_dim),
                         lambda i: (jnp.minimum(i, nblocks - 1), 0)),
            pl.BlockSpec((code_dim, input_dim), lambda i: (0, 0)),
            pl.BlockSpec((code_dim, code_dim), lambda i: (0, 0)),
        ],
        out_specs=pl.BlockSpec((block_rows, code_dim),
                               lambda i: (jnp.maximum(i - 1, 0), 0)),
        out_shape=jax.ShapeDtypeStruct((batch, code_dim), jnp.float32),
        scratch_shapes=[
            pltpu.VMEM((2, code_dim, block_rows), jnp.float32),
            pltpu.VMEM((1, block_rows), jnp.int32),
            pltpu.VMEM((1, block_rows), jnp.int32),
            pltpu.VMEM((1, block_rows), jnp.int32),
            pltpu.VMEM((1, block_rows), jnp.int32),
        ],
    )(x, z_prev, W, S)


# speedup vs baseline: 1.6207x; 1.1931x over previous
import functools

import jax
import jax.numpy as jnp
from jax.experimental import pallas as pl
from jax.experimental.pallas import tpu as pltpu

_K = 64


def _matmul_into(x_ref, z_ref, w_ref, s_ref, buf):
    updt = jax.lax.dot_general(
        w_ref[...], x_ref[...], (((1,), (1,)), ((), ())),
        preferred_element_type=jnp.float32)
    updt = updt + jax.lax.dot_general(
        s_ref[...], z_ref[...], (((1,), (1,)), ((), ())),
        preferred_element_type=jnp.float32)
    buf[...] = updt


def _tree_add(parts):
    while len(parts) > 1:
        parts = [a + b for a, b in zip(parts[::2], parts[1::2])]
    return parts[0]


_CHUNKS = 8


def _count_rows(bits, mid):
    step = bits.shape[0] // _CHUNKS
    return _tree_add([
        jnp.sum((bits[c * step:(c + 1) * step] >= mid).astype(jnp.int32),
                axis=0, keepdims=True) for c in range(_CHUNKS)])


def _probe(bits, state, mid):
    lo, hi, cnt_lo, cnt_hi = state
    mid = jnp.clip(mid, lo + 1, jnp.maximum(hi - 1, lo + 1))
    cnt = _count_rows(bits, mid)
    ge = cnt >= _K
    return (jnp.where(ge, mid, lo), jnp.where(ge, hi, mid),
            jnp.where(ge, cnt, cnt_lo), jnp.where(ge, cnt_hi, cnt))


def _interp_probe(bits, state):
    lo, hi, cnt_lo, cnt_hi = state
    span = (hi - lo).astype(jnp.float32)
    frac = (cnt_lo - _K).astype(jnp.float32) / jnp.maximum(
        cnt_lo - cnt_hi, 1).astype(jnp.float32)
    return _probe(bits, state, lo + (span * frac).astype(jnp.int32))


def _bisect_probe(bits, state):
    lo, hi, _, _ = state
    return _probe(bits, state, lo + ((hi - lo) >> 1))


def _unconverged(refs):
    lo, hi, cnt_lo, _ = refs
    return jnp.any((cnt_lo[...] != _K) & (hi[...] > lo[...] + 1))


def _store(refs, state):
    for r, v in zip(refs, state):
        r[...] = v


def _load(refs):
    return tuple(r[...] for r in refs)


def _select_store(buf, o_ref, refs):
    updt = buf[...]
    bits = jax.lax.bitcast_convert_type(updt, jnp.int32) & jnp.int32(0x7FFFFFFF)
    cols, rows = updt.shape
    step = cols // _CHUNKS
    bmax = functools.reduce(jnp.maximum, [
        jnp.max(bits[c * step:(c + 1) * step], axis=0, keepdims=True)
        for c in range(_CHUNKS)])
    state = (jnp.zeros((1, rows), jnp.int32),
             bmax + 1,
             jnp.full((1, rows), cols, jnp.int32),
             jnp.zeros((1, rows), jnp.int32))
    for _ in range(14):
        state = _interp_probe(bits, state)
    _store(refs, state)

    for extra in (5, 5):
        @pl.when(_unconverged(refs))
        def _(extra=extra):
            st = _load(refs)
            for _ in range(extra):
                st = _interp_probe(bits, st)
            _store(refs, st)

    @pl.when(_unconverged(refs))
    def _():
        st = jax.lax.fori_loop(
            0, 34, lambda _, s: _bisect_probe(bits, s), _load(refs))
        _store(refs, st)

    o_ref[...] = jnp.where(bits >= refs[0][...], updt, 0.0).T


def _pipelined_block(x_ref, z_ref, w_ref, s_ref, o_ref, buf, lo_r, hi_r,
                     cl_r, ch_r, *, nblocks):
    i = pl.program_id(0)
    _select_store(buf.at[(i + 1) % 2], o_ref, (lo_r, hi_r, cl_r, ch_r))
    _matmul_into(x_ref, z_ref, w_ref, s_ref, buf.at[i % 2])


@functools.partial(jax.jit, static_argnames=("block_rows",))
def kernel(x, z_prev, W, S, block_rows: int = 256):
    batch, input_dim = x.shape
    code_dim = W.shape[0]
    nblocks = batch // block_rows
    grid = (nblocks + 1,)
    return pl.pallas_call(
        functools.partial(_pipelined_block, nblocks=nblocks),
        grid=grid,
        in_specs=[
            pl.BlockSpec((block_rows, input_dim),
                         lambda i: (jnp.minimum(i, nblocks - 1), 0)),
            pl.BlockSpec((block_rows, code_dim),
                         lambda i: (jnp.minimum(i, nblocks - 1), 0)),
            pl.BlockSpec((code_dim, input_dim), lambda i: (0, 0)),
            pl.BlockSpec((code_dim, code_dim), lambda i: (0, 0)),
        ],
        out_specs=pl.BlockSpec((block_rows, code_dim),
                               lambda i: (jnp.maximum(i - 1, 0), 0)),
        out_shape=jax.ShapeDtypeStruct((batch, code_dim), jnp.float32),
        scratch_shapes=[
            pltpu.VMEM((2, code_dim, block_rows), jnp.float32),
            pltpu.VMEM((1, block_rows), jnp.int32),
            pltpu.VMEM((1, block_rows), jnp.int32),
            pltpu.VMEM((1, block_rows), jnp.int32),
            pltpu.VMEM((1, block_rows), jnp.int32),
        ],
    )(x, z_prev, W, S)


# speedup vs baseline: 2.5762x; 1.5896x over previous
import functools

import jax
import jax.numpy as jnp
from jax.experimental import pallas as pl
from jax.experimental.pallas import tpu as pltpu

_K = 64


def _matmul_into(x_ref, z_ref, w_ref, s_ref, buf):
    upd = jax.lax.dot_general(
        x_ref[...], w_ref[...], (((1,), (1,)), ((), ())),
        preferred_element_type=jnp.float32)
    upd = upd + jax.lax.dot_general(
        z_ref[...], s_ref[...], (((1,), (1,)), ((), ())),
        preferred_element_type=jnp.float32)
    buf[...] = upd


def _select_store(buf, o_ref):
    upd = buf[...]
    bits = jax.lax.bitcast_convert_type(upd, jnp.int32) & jnp.int32(0x7FFFFFFF)
    rows = upd.shape[0]
    t = jnp.zeros((rows, 1), jnp.int32)
    for b in range(30, -1, -1):
        cand = t | jnp.int32(1 << b)
        cnt = jnp.sum((bits >= cand).astype(jnp.int32), axis=1, keepdims=True)
        t = jnp.where(cnt >= _K, cand, t)
    o_ref[...] = jnp.where(bits >= t, upd, 0.0)


def _pipelined_block(x_ref, z_ref, w_ref, s_ref, o_ref, buf, *, nblocks):
    i = pl.program_id(0)
    _select_store(buf.at[(i + 1) % 2], o_ref)
    _matmul_into(x_ref, z_ref, w_ref, s_ref, buf.at[i % 2])


@functools.partial(jax.jit, static_argnames=("block_rows",))
def kernel(x, z_prev, W, S, block_rows: int = 256):
    batch, input_dim = x.shape
    code_dim = W.shape[0]
    nblocks = batch // block_rows
    grid = (nblocks + 1,)
    return pl.pallas_call(
        functools.partial(_pipelined_block, nblocks=nblocks),
        grid=grid,
        in_specs=[
            pl.BlockSpec((block_rows, input_dim),
                         lambda i: (jnp.minimum(i, nblocks - 1), 0)),
            pl.BlockSpec((block_rows, code_dim),
                         lambda i: (jnp.minimum(i, nblocks - 1), 0)),
            pl.BlockSpec((code_dim, input_dim), lambda i: (0, 0)),
            pl.BlockSpec((code_dim, code_dim), lambda i: (0, 0)),
        ],
        out_specs=pl.BlockSpec((block_rows, code_dim),
                               lambda i: (jnp.maximum(i - 1, 0), 0)),
        out_shape=jax.ShapeDtypeStruct((batch, code_dim), jnp.float32),
        scratch_shapes=[
            pltpu.VMEM((2, block_rows, code_dim), jnp.float32),
        ],
    )(x, z_prev, W, S)


# speedup vs baseline: 2.6229x; 1.0181x over previous
import functools

import jax
import jax.numpy as jnp
from jax.experimental import pallas as pl
from jax.experimental.pallas import tpu as pltpu

_K = 64


def _matmul_into(x_ref, z_ref, w_ref, s_ref, buf):
    upd = jax.lax.dot_general(
        x_ref[...], w_ref[...], (((1,), (1,)), ((), ())),
        preferred_element_type=jnp.float32)
    upd = upd + jax.lax.dot_general(
        z_ref[...], s_ref[...], (((1,), (1,)), ((), ())),
        preferred_element_type=jnp.float32)
    buf[...] = upd


def _select_store(buf, o_ref):
    upd = buf[...]
    bits = jax.lax.bitcast_convert_type(upd, jnp.int32) & jnp.int32(0x7FFFFFFF)
    rows = upd.shape[0]
    t = jnp.zeros((rows, 1), jnp.int32)
    for b in range(30, -1, -1):
        cand = t | jnp.int32(1 << b)
        cnt = jnp.sum((bits >= cand).astype(jnp.int32), axis=1, keepdims=True)
        t = jnp.where(cnt >= _K, cand, t)
    o_ref[...] = jnp.where(bits >= t, upd, 0.0)


def _pipelined_block(x_ref, z_ref, w_ref, s_ref, o_ref, buf, *, nblocks):
    i = pl.program_id(0)
    _select_store(buf.at[(i + 1) % 2], o_ref)
    _matmul_into(x_ref, z_ref, w_ref, s_ref, buf.at[i % 2])


@functools.partial(jax.jit, static_argnames=("block_rows",))
def kernel(x, z_prev, W, S, block_rows: int = 512):
    batch, input_dim = x.shape
    code_dim = W.shape[0]
    nblocks = batch // block_rows
    grid = (nblocks + 1,)
    return pl.pallas_call(
        functools.partial(_pipelined_block, nblocks=nblocks),
        grid=grid,
        in_specs=[
            pl.BlockSpec((block_rows, input_dim),
                         lambda i: (jnp.minimum(i, nblocks - 1), 0)),
            pl.BlockSpec((block_rows, code_dim),
                         lambda i: (jnp.minimum(i, nblocks - 1), 0)),
            pl.BlockSpec((code_dim, input_dim), lambda i: (0, 0)),
            pl.BlockSpec((code_dim, code_dim), lambda i: (0, 0)),
        ],
        out_specs=pl.BlockSpec((block_rows, code_dim),
                               lambda i: (jnp.maximum(i - 1, 0), 0)),
        out_shape=jax.ShapeDtypeStruct((batch, code_dim), jnp.float32),
        scratch_shapes=[
            pltpu.VMEM((2, block_rows, code_dim), jnp.float32),
        ],
    )(x, z_prev, W, S)
